# Initial kernel scaffold; baseline (speedup 1.0000x reference)
import jax
import jax.numpy as jnp
from jax.experimental import pallas as pl


def kernel(x, fixed_edge_index, fixed_edge_attr, W, mix_logit):
    raise NotImplementedError("write your pallas kernel here")


# speedup vs baseline: 17.8617x; 17.8617x over previous
import functools
import math

import jax
import jax.numpy as jnp
from jax.experimental import pallas as pl
from jax.experimental.pallas import tpu as pltpu

_KTOP = 16
_RB = 512


def _embed_kernel(x_ref, wt_ref, e_ref):
    xa = jnp.mean(x_ref[...], axis=1)
    y = jax.lax.dot_general(
        xa, wt_ref[...], (((1,), (0,)), ((), ())),
        precision=jax.lax.Precision.DEFAULT,
        preferred_element_type=jnp.float32)
    nrm = jnp.sqrt(jnp.sum(y * y, axis=1, keepdims=True))
    e_ref[...] = y / jnp.maximum(nrm, 1e-12)


def _topk_kernel(n, k, e_blk_ref, et_ref, alpha_ref, vals_ref, idx_ref):
    n_pad = et_ref.shape[1]
    sim = jax.lax.dot_general(
        e_blk_ref[...], et_ref[...], (((1,), (0,)), ((), ())),
        precision=jax.lax.Precision.DEFAULT,
        preferred_element_type=jnp.float32)
    colv = jax.lax.broadcasted_iota(jnp.int32, (1, n_pad), 1)
    sim = jnp.where(colv < n, sim, -jnp.inf)
    alpha = alpha_ref[0, 0]
    for i in range(k + 1):
        m = jnp.max(sim, axis=1, keepdims=True)
        cand = jnp.where(sim == m, colv, n_pad + 1)
        idx = jnp.min(cand, axis=1, keepdims=True)
        if i > 0:
            vals_ref[:, i - 1:i] = m * alpha
            idx_ref[:, i - 1:i] = idx
        if i < k:
            sim = jnp.where(cand == idx, -jnp.inf, sim)


def _scale_kernel(a_ref, s_ref, o_ref):
    o_ref[...] = a_ref[...] * s_ref[0, 0]


def kernel(x, fixed_edge_index, fixed_edge_attr, W, mix_logit):
    n, t, h = x.shape
    d = W.shape[0]
    k = min(_KTOP, n - 1)
    rb = _RB
    n_pad = ((n + rb - 1) // rb) * rb
    nb = n_pad // rb

    x_pad = jnp.pad(x, ((0, n_pad - n), (0, 0), (0, 0)))
    wt = W.T

    e = pl.pallas_call(
        _embed_kernel,
        grid=(nb,),
        in_specs=[
            pl.BlockSpec((rb, t, h), lambda i: (i, 0, 0)),
            pl.BlockSpec((h, d), lambda i: (0, 0)),
        ],
        out_specs=pl.BlockSpec((rb, d), lambda i: (i, 0)),
        out_shape=jax.ShapeDtypeStruct((n_pad, d), jnp.float32),
    )(x_pad, wt)

    et = e.T
    alpha = jax.nn.sigmoid(mix_logit).reshape(1, 1)

    vals, idxs = pl.pallas_call(
        functools.partial(_topk_kernel, n, k),
        grid=(nb,),
        in_specs=[
            pl.BlockSpec((rb, d), lambda i: (i, 0)),
            pl.BlockSpec((d, n_pad), lambda i: (0, 0)),
            pl.BlockSpec((1, 1), lambda i: (0, 0)),
        ],
        out_specs=[
            pl.BlockSpec((rb, k), lambda i: (i, 0)),
            pl.BlockSpec((rb, k), lambda i: (i, 0)),
        ],
        out_shape=[
            jax.ShapeDtypeStruct((n_pad, k), jnp.float32),
            jax.ShapeDtypeStruct((n_pad, k), jnp.int32),
        ],
    )(e, et, alpha)

    e_fixed = fixed_edge_attr.shape[0]
    flat = fixed_edge_attr.reshape(-1)
    pad_f = (-e_fixed) % 128
    flat = jnp.pad(flat, (0, pad_f)).reshape(-1, 128)
    one_minus_alpha = (1.0 - jax.nn.sigmoid(mix_logit)).reshape(1, 1)
    fixed_scaled = pl.pallas_call(
        _scale_kernel,
        in_specs=[
            pl.BlockSpec(flat.shape, lambda: (0, 0)),
            pl.BlockSpec((1, 1), lambda: (0, 0)),
        ],
        out_specs=pl.BlockSpec(flat.shape, lambda: (0, 0)),
        out_shape=jax.ShapeDtypeStruct(flat.shape, jnp.float32),
    )(flat, one_minus_alpha)
    fixed_scaled = fixed_scaled.reshape(-1)[:e_fixed].reshape(-1, 1)

    src = jnp.repeat(jnp.arange(n, dtype=jnp.int32), k)
    dst = idxs[:n].reshape(-1)
    dyn_edge_index = jnp.stack([src, dst], axis=0)
    dyn_edge_attr = vals[:n].reshape(-1, 1)

    combined_edge_index = jnp.concatenate([fixed_edge_index, dyn_edge_index], axis=1)
    combined_edge_attr = jnp.concatenate([fixed_scaled, dyn_edge_attr], axis=0)
    return combined_edge_index, combined_edge_attr


# speedup vs baseline: 19.2143x; 1.0757x over previous
import functools
import math

import jax
import jax.numpy as jnp
from jax.experimental import pallas as pl
from jax.experimental.pallas import tpu as pltpu

_KTOP = 16
_RB = 256
_W = 256
_P = 4


def _embed_kernel(x_ref, wt_ref, e_ref):
    xa = jnp.mean(x_ref[...], axis=1)
    y = jax.lax.dot_general(
        xa, wt_ref[...], (((1,), (0,)), ((), ())),
        precision=jax.lax.Precision.DEFAULT,
        preferred_element_type=jnp.float32)
    nrm = jnp.sqrt(jnp.sum(y * y, axis=1, keepdims=True))
    e_ref[...] = y / jnp.maximum(nrm, 1e-12)


def _extract_topk(vmat, imat, k, vals_ref, idx_ref, alpha, write):
    big = jnp.int32(2 ** 30)
    last = None
    outs = []
    for i in range(k + 1):
        m = jnp.max(vmat, axis=1, keepdims=True)
        cand = jnp.where(vmat == m, imat, big)
        gidx = jnp.min(cand, axis=1, keepdims=True)
        if i > 0:
            outs.append((m, gidx))
        if i < k:
            vmat = jnp.where(cand == gidx, -jnp.inf, vmat)
        last = m
    if write:
        for i, (m, gidx) in enumerate(outs):
            vals_ref[:, i:i + 1] = m * alpha
            idx_ref[:, i:i + 1] = gidx
        return None
    return outs, last


def _topk_kernel(n, k, e_blk_ref, et_ref, alpha_ref, vals_ref, idx_ref):
    rb = e_blk_ref.shape[0]
    n_pad = et_ref.shape[1]
    nc = n_pad // _W
    sim = jax.lax.dot_general(
        e_blk_ref[...], et_ref[...], (((1,), (0,)), ((), ())),
        precision=jax.lax.Precision.DEFAULT,
        preferred_element_type=jnp.float32)
    colv = jax.lax.broadcasted_iota(jnp.int32, (1, n_pad), 1)
    sim = jnp.where(colv < n, sim, -jnp.inf)
    alpha = alpha_ref[0, 0]

    s3 = sim.reshape(rb, nc, _W)
    ci3 = jax.lax.broadcasted_iota(jnp.int32, (1, nc, _W), 1)
    wi = jax.lax.broadcasted_iota(jnp.int32, (1, _W), 1)
    big = jnp.int32(2 ** 30)
    cvals, cidx = [], []
    for _ in range(_P):
        m = jnp.max(s3, axis=1)
        cand = jnp.where(s3 == m[:, None, :], ci3, big)
        a = jnp.min(cand, axis=1)
        cvals.append(m)
        cidx.append(a * _W + wi)
        s3 = jnp.where(cand == a[:, None, :], -jnp.inf, s3)
    leftover = jnp.max(jnp.max(s3, axis=1), axis=1, keepdims=True)
    vmat = jnp.concatenate(cvals, axis=1)
    imat = jnp.concatenate(cidx, axis=1)
    outs, last = _extract_topk(vmat, imat, k, None, None, alpha, write=False)
    fastok = jnp.all(last > leftover)

    @pl.when(fastok)
    def _():
        for i, (m, gidx) in enumerate(outs):
            vals_ref[:, i:i + 1] = m * alpha
            idx_ref[:, i:i + 1] = gidx

    @pl.when(jnp.logical_not(fastok))
    def _():
        _extract_topk(sim, jnp.broadcast_to(colv, sim.shape), k,
                      vals_ref, idx_ref, alpha, write=True)


def _scale_kernel(a_ref, s_ref, o_ref):
    o_ref[...] = a_ref[...] * s_ref[0, 0]


def kernel(x, fixed_edge_index, fixed_edge_attr, W, mix_logit):
    n, t, h = x.shape
    d = W.shape[0]
    k = min(_KTOP, n - 1)
    rb = _RB
    n_pad = ((n + rb - 1) // rb) * rb
    nb = n_pad // rb

    x_pad = jnp.pad(x, ((0, n_pad - n), (0, 0), (0, 0)))
    wt = W.T

    e = pl.pallas_call(
        _embed_kernel,
        grid=(nb,),
        in_specs=[
            pl.BlockSpec((rb, t, h), lambda i: (i, 0, 0)),
            pl.BlockSpec((h, d), lambda i: (0, 0)),
        ],
        out_specs=pl.BlockSpec((rb, d), lambda i: (i, 0)),
        out_shape=jax.ShapeDtypeStruct((n_pad, d), jnp.float32),
    )(x_pad, wt)

    et = e.T
    alpha = jax.nn.sigmoid(mix_logit).reshape(1, 1)

    vals, idxs = pl.pallas_call(
        functools.partial(_topk_kernel, n, k),
        grid=(nb,),
        in_specs=[
            pl.BlockSpec((rb, d), lambda i: (i, 0)),
            pl.BlockSpec((d, n_pad), lambda i: (0, 0)),
            pl.BlockSpec((1, 1), lambda i: (0, 0)),
        ],
        out_specs=[
            pl.BlockSpec((rb, k), lambda i: (i, 0)),
            pl.BlockSpec((rb, k), lambda i: (i, 0)),
        ],
        out_shape=[
            jax.ShapeDtypeStruct((n_pad, k), jnp.float32),
            jax.ShapeDtypeStruct((n_pad, k), jnp.int32),
        ],
    )(e, et, alpha)

    e_fixed = fixed_edge_attr.shape[0]
    flat = fixed_edge_attr.reshape(-1)
    pad_f = (-e_fixed) % 128
    flat = jnp.pad(flat, (0, pad_f)).reshape(-1, 128)
    one_minus_alpha = (1.0 - jax.nn.sigmoid(mix_logit)).reshape(1, 1)
    fixed_scaled = pl.pallas_call(
        _scale_kernel,
        in_specs=[
            pl.BlockSpec(flat.shape, lambda: (0, 0)),
            pl.BlockSpec((1, 1), lambda: (0, 0)),
        ],
        out_specs=pl.BlockSpec(flat.shape, lambda: (0, 0)),
        out_shape=jax.ShapeDtypeStruct(flat.shape, jnp.float32),
    )(flat, one_minus_alpha)
    fixed_scaled = fixed_scaled.reshape(-1)[:e_fixed].reshape(-1, 1)

    src = jnp.repeat(jnp.arange(n, dtype=jnp.int32), k)
    dst = idxs[:n].reshape(-1)
    dyn_edge_index = jnp.stack([src, dst], axis=0)
    dyn_edge_attr = vals[:n].reshape(-1, 1)

    combined_edge_index = jnp.concatenate([fixed_edge_index, dyn_edge_index], axis=1)
    combined_edge_attr = jnp.concatenate([fixed_scaled, dyn_edge_attr], axis=0)
    return combined_edge_index, combined_edge_attr
